# baseline (speedup 1.0000x reference)
import functools
import math

import jax
import jax.numpy as jnp
from jax import lax
from jax.experimental import pallas as pl
from jax.experimental.pallas import tpu as pltpu
from jax.experimental.pallas import tpu_sc as plsc

EMB_DEPTH = 32
ROWS = 16384
COLS = 200
B = ROWS * COLS
VOC = 1000000
YROWS = B * EMB_DEPTH // 128
TROWS = VOC * EMB_DEPTH // 128
SCALE = math.sqrt(float(EMB_DEPTH))

_info = plsc.get_sparse_core_info()
NC = _info.num_cores
NS = _info.num_subcores
L = _info.num_lanes
NW = NC * NS

_mesh = plsc.VectorSubcoreMesh(core_axis_name="c", subcore_axis_name="s")


def _wid():
    return lax.axis_index("s") * NC + lax.axis_index("c")


PK = 160
PCHUNKS = VOC // PK
PSLOTS = 196
PPAIR = 194 // 2


@functools.partial(
    pl.kernel,
    mesh=_mesh,
    out_type=jax.ShapeDtypeStruct((TROWS, 128), jnp.float32),
    scratch_types=[
        pltpu.VMEM((2, PK, EMB_DEPTH), jnp.float32),
        pltpu.VMEM((2, PK // 4, 128), jnp.float32),
        pltpu.SemaphoreType.DMA((2,)),
        pltpu.SemaphoreType.DMA((2,)),
    ],
    compiler_params=pltpu.CompilerParams(use_tc_tiling_on_sc=True),
)
def _detile(tab_hbm, ytab_hbm, v1, v2, si, so):
    wid = _wid()

    def c_of(t):
        return wid + NW * t

    def in_start(t, b):
        off = pl.multiple_of(c_of(t) * PK, 8)
        pltpu.async_copy(
            tab_hbm.at[pl.ds(off, PK), :], v1.at[b], si.at[b])

    def in_wait(b):
        pltpu.make_async_copy(
            tab_hbm.at[pl.ds(0, PK), :], v1.at[b], si.at[b]).wait()

    def out_start(t, b):
        off = pl.multiple_of(c_of(t) * (PK // 4), 8)
        pltpu.async_copy(
            v2.at[b], ytab_hbm.at[pl.ds(off, PK // 4), :], so.at[b])

    def out_wait(b):
        pltpu.make_async_copy(
            v2.at[b], ytab_hbm.at[pl.ds(0, PK // 4), :], so.at[b]).wait()

    def repack(b):
        @plsc.parallel_loop(0, PK // 4, 1, unroll=2)
        def _(m):
            for jj in range(4):
                for k in range(2):
                    v2[b, m, pl.ds(32 * jj + 16 * k, L)] = (
                        v1[b, 4 * m + jj, pl.ds(16 * k, L)])

    in_start(0, 0)
    in_start(1, 1)

    def pair_body(p, carry):
        for b in (0, 1):
            t = 2 * p + b
            in_wait(b)

            @pl.when(t >= 2)
            def _():
                out_wait(b)

            repack(b)

            @pl.when(c_of(t + 2) < PCHUNKS)
            def _():
                in_start(t + 2, b)

            out_start(t, b)
        return carry

    lax.fori_loop(0, PPAIR, pair_body, 0)

    in_wait(0)
    out_wait(0)
    repack(0)
    out_start(PSLOTS - 2, 0)

    @pl.when(c_of(PSLOTS - 1) < PCHUNKS)
    def _():
        in_wait(1)
        out_wait(1)
        repack(1)
        out_start(PSLOTS - 1, 1)

    out_wait(0)
    out_wait(1)


RPW = ROWS // NW
RCH = 4
NLK = RCH * COLS
NYR = NLK * EMB_DEPTH // 128
NCHUNK = RPW // RCH
NPAIR = NCHUNK // 2


@functools.partial(
    pl.kernel,
    mesh=_mesh,
    out_type=jax.ShapeDtypeStruct((YROWS, 128), jnp.float32),
    scratch_types=[
        pltpu.VMEM((2, RCH, COLS), jnp.int32),
        pltpu.VMEM((2, NLK, EMB_DEPTH), jnp.float32),
        pltpu.VMEM((2, NYR, 128), jnp.float32),
        pltpu.SemaphoreType.DMA((2,)),
        pltpu.SemaphoreType.DMA((2,)),
        pltpu.SemaphoreType.DMA((2,)),
    ],
    compiler_params=pltpu.CompilerParams(use_tc_tiling_on_sc=False),
)
def _gather_scale(x_hbm, table_hbm, y_hbm, idx_v, g_v, s_v, si, sg, so):
    wid = _wid()
    rbase = wid * RPW

    def r0(g):
        return rbase + g * RCH

    def y0(g):
        return r0(g) * (COLS * EMB_DEPTH // 128)

    def idx_start(g, b):
        pltpu.async_copy(x_hbm.at[pl.ds(r0(g), RCH), :], idx_v.at[b], si.at[b])

    def idx_wait(b):
        pltpu.make_async_copy(
            x_hbm.at[pl.ds(0, RCH), :], idx_v.at[b], si.at[b]).wait()

    def gather_start(b):
        for r in range(RCH):
            pltpu.async_copy(
                table_hbm.at[idx_v.at[b, r]],
                g_v.at[b, pl.ds(r * COLS, COLS), :], sg.at[b])

    def gather_wait(b):
        for r in range(RCH):
            pltpu.make_async_copy(
                table_hbm.at[idx_v.at[b, r]],
                g_v.at[b, pl.ds(r * COLS, COLS), :], sg.at[b]).wait()

    def store_start(g, b):
        pltpu.async_copy(s_v.at[b], y_hbm.at[pl.ds(y0(g), NYR), :], so.at[b])

    def store_wait(b):
        pltpu.make_async_copy(
            s_v.at[b], y_hbm.at[pl.ds(0, NYR), :], so.at[b]).wait()

    def scale_repack(b):
        @plsc.parallel_loop(0, NYR, 1, unroll=2)
        def _(j):
            for jj in range(4):
                for k in range(2):
                    s_v[b, j, pl.ds(32 * jj + 16 * k, L)] = (
                        g_v[b, 4 * j + jj, pl.ds(16 * k, L)] * SCALE)

    idx_start(0, 0)
    idx_start(1, 1)
    idx_wait(0)
    gather_start(0)

    def pair_body(p, carry):
        for b in (0, 1):
            g = 2 * p + b
            o = 1 - b
            gather_wait(b)

            @pl.when(g + 2 < NCHUNK)
            def _():
                idx_start(g + 2, b)

            @pl.when(g + 1 < NCHUNK)
            def _():
                @pl.when(g >= 1)
                def _():
                    store_wait(o)
                idx_wait(o)
                gather_start(o)

            scale_repack(b)
            store_start(g, b)
        return carry

    lax.fori_loop(0, NPAIR, pair_body, 0)
    store_wait(0)
    store_wait(1)


CNN = 256
CIN = CNN * EMB_DEPTH // 128
LPW = B // NW
CCHUNK = LPW // CNN
CPAIR = CCHUNK // 2


@functools.partial(
    pl.kernel,
    mesh=_mesh,
    out_type=jax.ShapeDtypeStruct((B, EMB_DEPTH), jnp.float32),
    scratch_types=[
        pltpu.VMEM((2, CIN, 128), jnp.float32),
        pltpu.VMEM((2, CNN, EMB_DEPTH), jnp.float32),
        pltpu.SemaphoreType.DMA((2,)),
        pltpu.SemaphoreType.DMA((2,)),
    ],
    compiler_params=pltpu.CompilerParams(use_tc_tiling_on_sc=True),
)
def _repack(y_hbm, out_hbm, v1, v2, si, so):
    wid = _wid()
    nbase = wid * LPW

    def n0(t):
        return nbase + t * CNN

    def in_start(t, b):
        off = pl.multiple_of(n0(t) * EMB_DEPTH // 128, 8)
        pltpu.async_copy(
            y_hbm.at[pl.ds(off, CIN), :], v1.at[b], si.at[b])

    def in_wait(b):
        pltpu.make_async_copy(
            y_hbm.at[pl.ds(0, CIN), :], v1.at[b], si.at[b]).wait()

    def out_start(t, b):
        off = pl.multiple_of(n0(t), 8)
        pltpu.async_copy(
            v2.at[b], out_hbm.at[pl.ds(off, CNN), :], so.at[b])

    def out_wait(b):
        pltpu.make_async_copy(
            v2.at[b], out_hbm.at[pl.ds(0, CNN), :], so.at[b]).wait()

    def repack(b):
        @plsc.parallel_loop(0, CIN, 1, unroll=2)
        def _(j):
            for jj in range(4):
                for k in range(2):
                    v2[b, 4 * j + jj, pl.ds(16 * k, L)] = (
                        v1[b, j, pl.ds(32 * jj + 16 * k, L)])

    in_start(0, 0)
    in_start(1, 1)

    def pair_body(p, carry):
        for b in (0, 1):
            t = 2 * p + b
            in_wait(b)

            @pl.when(t >= 2)
            def _():
                out_wait(b)

            repack(b)

            @pl.when(t + 2 < CCHUNK)
            def _():
                in_start(t + 2, b)

            out_start(t, b)
        return carry

    lax.fori_loop(0, CPAIR, pair_body, 0)
    out_wait(0)
    out_wait(1)


def kernel(x, table):
    ytab = _detile(table)
    table_lin = ytab.reshape(VOC, EMB_DEPTH)
    y = _gather_scale(x, table_lin)
    return _repack(y).reshape(ROWS, COLS, EMB_DEPTH)


# speedup vs baseline: 1.6025x; 1.6025x over previous
import functools
import math

import jax
import jax.numpy as jnp
from jax import lax
from jax.experimental import pallas as pl
from jax.experimental.pallas import tpu as pltpu
from jax.experimental.pallas import tpu_sc as plsc

EMB_DEPTH = 32
ROWS = 16384
COLS = 200
SCALE = math.sqrt(float(EMB_DEPTH))

_info = plsc.get_sparse_core_info()
NC = _info.num_cores
NS = _info.num_subcores
L = _info.num_lanes
NW = NC * NS
RPW = ROWS // NW
RCH = 8
NCHUNK = RPW // RCH
NPAIR = NCHUNK // 2

assert ROWS % NW == 0 and RPW % RCH == 0 and NCHUNK % 2 == 0

_mesh = plsc.VectorSubcoreMesh(core_axis_name="c", subcore_axis_name="s")


@functools.partial(
    pl.kernel,
    mesh=_mesh,
    out_type=jax.ShapeDtypeStruct((ROWS, COLS, 128), jnp.float32),
    scratch_types=[
        pltpu.VMEM((2, RCH, COLS), jnp.int32),
        pltpu.VMEM((2, RCH, COLS, EMB_DEPTH), jnp.float32),
        pltpu.SemaphoreType.DMA((2,)),
        pltpu.SemaphoreType.DMA((2,)),
        pltpu.SemaphoreType.DMA((2,)),
    ],
    compiler_params=pltpu.CompilerParams(use_tc_tiling_on_sc=False),
)
def _embed(x_hbm, table_hbm, out_hbm, idx_v, rows_v, si, sg, so):
    wid = lax.axis_index("s") * NC + lax.axis_index("c")
    rbase = wid * RPW

    def r0(g):
        return rbase + g * RCH

    def idx_start(g, b):
        pltpu.async_copy(x_hbm.at[pl.ds(r0(g), RCH), :], idx_v.at[b], si.at[b])

    def idx_wait(b):
        pltpu.make_async_copy(
            x_hbm.at[pl.ds(0, RCH), :], idx_v.at[b], si.at[b]).wait()

    def gather_start(b):
        for r in range(RCH):
            pltpu.async_copy(
                table_hbm.at[idx_v.at[b, r]], rows_v.at[b, r], sg.at[b])

    def gather_wait(b):
        for r in range(RCH):
            pltpu.make_async_copy(
                table_hbm.at[idx_v.at[b, r]], rows_v.at[b, r], sg.at[b]).wait()

    def store_start(g, b):
        pltpu.async_copy(
            rows_v.at[b],
            out_hbm.at[pl.ds(r0(g), RCH), :, pl.ds(0, EMB_DEPTH)], so.at[b])

    def store_wait(b):
        pltpu.make_async_copy(
            rows_v.at[b],
            out_hbm.at[pl.ds(0, RCH), :, pl.ds(0, EMB_DEPTH)], so.at[b]).wait()

    def scale_rows(b):
        for r in range(RCH):
            @plsc.parallel_loop(0, COLS, 1, unroll=8)
            def _(c):
                rows_v[b, r, c, pl.ds(0, L)] = rows_v[b, r, c, pl.ds(0, L)] * SCALE
                rows_v[b, r, c, pl.ds(L, L)] = rows_v[b, r, c, pl.ds(L, L)] * SCALE

    idx_start(0, 0)
    idx_start(1, 1)
    idx_wait(0)
    gather_start(0)

    def pair_body(p, carry):
        for b in (0, 1):
            g = 2 * p + b
            o = 1 - b
            gather_wait(b)

            @pl.when(g + 2 < NCHUNK)
            def _():
                idx_start(g + 2, b)

            @pl.when(g + 1 < NCHUNK)
            def _():
                @pl.when(g >= 1)
                def _():
                    store_wait(o)
                idx_wait(o)
                gather_start(o)

            scale_rows(b)
            store_start(g, b)
        return carry

    lax.fori_loop(0, NPAIR, pair_body, 0)
    store_wait(0)
    store_wait(1)


def kernel(x, table):
    return _embed(x, table)[..., :EMB_DEPTH]


# speedup vs baseline: 1.6029x; 1.0002x over previous
import functools
import math

import jax
import jax.numpy as jnp
from jax import lax
from jax.experimental import pallas as pl
from jax.experimental.pallas import tpu as pltpu
from jax.experimental.pallas import tpu_sc as plsc

EMB_DEPTH = 32
ROWS = 16384
COLS = 200
SCALE = math.sqrt(float(EMB_DEPTH))

_info = plsc.get_sparse_core_info()
NC = _info.num_cores
NS = _info.num_subcores
L = _info.num_lanes
NW = NC * NS
RPW = ROWS // NW
RCH = 4
NCHUNK = RPW // RCH
NQUAD = NCHUNK // 4

assert ROWS % NW == 0 and RPW % RCH == 0 and NCHUNK % 4 == 0

_mesh = plsc.VectorSubcoreMesh(core_axis_name="c", subcore_axis_name="s")


@functools.partial(
    pl.kernel,
    mesh=_mesh,
    out_type=jax.ShapeDtypeStruct((ROWS, COLS, 128), jnp.float32),
    scratch_types=[
        pltpu.VMEM((4, RCH, COLS), jnp.int32),
        pltpu.VMEM((4, RCH, COLS, EMB_DEPTH), jnp.float32),
        pltpu.SemaphoreType.DMA((4,)),
        pltpu.SemaphoreType.DMA((4,)),
        pltpu.SemaphoreType.DMA((4,)),
    ],
    compiler_params=pltpu.CompilerParams(use_tc_tiling_on_sc=False),
)
def _embed(x_hbm, table_hbm, out_hbm, idx_v, rows_v, si, sg, so):
    wid = lax.axis_index("s") * NC + lax.axis_index("c")
    rbase = wid * RPW

    def r0(g):
        return rbase + g * RCH

    def idx_start(g, b):
        pltpu.async_copy(x_hbm.at[pl.ds(r0(g), RCH), :], idx_v.at[b], si.at[b])

    def idx_wait(b):
        pltpu.make_async_copy(
            x_hbm.at[pl.ds(0, RCH), :], idx_v.at[b], si.at[b]).wait()

    def gather_start(b):
        for r in range(RCH):
            pltpu.async_copy(
                table_hbm.at[idx_v.at[b, r]], rows_v.at[b, r], sg.at[b])

    def gather_wait(b):
        for r in range(RCH):
            pltpu.make_async_copy(
                table_hbm.at[idx_v.at[b, r]], rows_v.at[b, r], sg.at[b]).wait()

    def store_start(g, b):
        pltpu.async_copy(
            rows_v.at[b],
            out_hbm.at[pl.ds(r0(g), RCH), :, pl.ds(0, EMB_DEPTH)], so.at[b])

    def store_wait(b):
        pltpu.make_async_copy(
            rows_v.at[b],
            out_hbm.at[pl.ds(0, RCH), :, pl.ds(0, EMB_DEPTH)], so.at[b]).wait()

    def scale_rows(b):
        for r in range(RCH):
            @plsc.parallel_loop(0, COLS, 1, unroll=8)
            def _(c):
                rows_v[b, r, c, pl.ds(0, L)] = rows_v[b, r, c, pl.ds(0, L)] * SCALE
                rows_v[b, r, c, pl.ds(L, L)] = rows_v[b, r, c, pl.ds(L, L)] * SCALE

    for b in range(4):
        idx_start(b, b)
    for b in range(3):
        idx_wait(b)
        gather_start(b)

    def quad_body(p, carry):
        for b in (0, 1, 2, 3):
            g = 4 * p + b
            b3 = (b + 3) % 4
            gather_wait(b)

            @pl.when(g + 4 < NCHUNK)
            def _():
                idx_start(g + 4, b)

            @pl.when(g + 3 < NCHUNK)
            def _():
                @pl.when(g >= 1)
                def _():
                    store_wait(b3)
                idx_wait(b3)
                gather_start(b3)

            scale_rows(b)
            store_start(g, b)
        return carry

    lax.fori_loop(0, NQUAD, quad_body, 0)
    for b in range(4):
        store_wait(b)


def kernel(x, table):
    return _embed(x, table)[..., :EMB_DEPTH]
